# baseline (speedup 1.0000x reference)
import functools

import jax
import jax.numpy as jnp
from jax import lax
from jax.experimental import pallas as pl
from jax.experimental.pallas import tpu as pltpu
from jax.experimental.pallas import tpu_sc as plsc

B, N, M = 4, 16384, 1024
C_FEAT = 64
C_PREV = 64
C_OUT = 64

TILE_N = 1024


def _knn_body(xyzt_ref, xyzp_ref, iout_ref, wout_ref):
    q = xyzt_ref[0]
    k = xyzp_ref[0]

    d2 = jnp.zeros((M, TILE_N), jnp.float32)
    for c in range(3):
        diff = k[:, c:c + 1] - q[c:c + 1, :]
        d2 = d2 + diff * diff

    iota = jax.lax.broadcasted_iota(jnp.int32, (M, TILE_N), 0)
    bits = jax.lax.bitcast_convert_type(d2, jnp.int32) + (M // 2)
    keys = (bits & ~(M - 1)) | iota

    kcur = keys
    idxs = []
    invs = []
    tot = jnp.zeros((1, TILE_N), jnp.float32)
    for kk in range(3):
        fcur = jax.lax.bitcast_convert_type(kcur, jnp.float32)
        mkf = jnp.min(fcur, axis=0, keepdims=True)
        mk = jax.lax.bitcast_convert_type(mkf, jnp.int32)
        d2k = jax.lax.bitcast_convert_type(mk & ~(M - 1), jnp.float32)
        inv = 1.0 / (jnp.sqrt(d2k) + 1e-8)
        tot = tot + inv
        idxs.append(mk & (M - 1))
        invs.append(inv)
        if kk < 2:
            kcur = jnp.where(kcur == mk, jnp.int32(0x7F7FFFFF), kcur)

    b = pl.program_id(0)
    iout_ref[0] = jnp.concatenate(idxs, axis=0) + b * M
    rtot = 1.0 / tot
    wcat = jnp.concatenate([w * rtot for w in invs], axis=0)
    wout_ref[0] = jnp.transpose(wcat)


def _mlp_body(g_ref, wt_ref, feat_ref, w1a_ref, w1b_ref, b1_ref, out_ref):
    wt = wt_ref[0]
    interp = g_ref[0, 0, :, :C_PREV] * wt[:, 0:1]
    interp = interp + g_ref[0, 1, :, :C_PREV] * wt[:, 1:2]
    interp = interp + g_ref[0, 2, :, :C_PREV] * wt[:, 2:3]
    h = (jax.lax.dot_general(w1a_ref[...], interp, (((1,), (1,)), ((), ())),
                             preferred_element_type=jnp.float32)
         + jnp.dot(w1b_ref[...], feat_ref[0], preferred_element_type=jnp.float32)
         + b1_ref[...])
    out_ref[0] = jnp.maximum(h, 0.0)


def _sc_gather(table, idx_flat):
    info = plsc.get_sparse_core_info()
    nc, ns = info.num_cores, info.num_subcores
    nw = nc * ns
    total = B * 3 * N
    chunk = total // nw
    piece = 384

    mesh = plsc.VectorSubcoreMesh(core_axis_name="c", subcore_axis_name="s")

    @functools.partial(
        pl.kernel, mesh=mesh,
        out_type=jax.ShapeDtypeStruct((total, 128), jnp.float32),
        scratch_types=[
            pltpu.VMEM((chunk,), jnp.int32),
            pltpu.VMEM((piece, 128), jnp.float32),
            pltpu.VMEM((piece, 128), jnp.float32),
            pltpu.SemaphoreType.DMA,
            pltpu.SemaphoreType.DMA,
        ],
    )
    def gather_kernel(table_hbm, idx_hbm, out_hbm, idx_v, rows0, rows1, sem0, sem1):
        wid = lax.axis_index("s") * nc + lax.axis_index("c")
        base = wid * chunk
        pltpu.sync_copy(idx_hbm.at[pl.ds(base, chunk)], idx_v)
        bufs = (rows0, rows1)
        sems = (sem0, sem1)
        nump = chunk // piece
        cp = pltpu.async_copy(
            table_hbm.at[idx_v.at[pl.ds(0, piece)]], bufs[0], sems[0])
        for p in range(nump):
            cur = p % 2
            cp.wait()
            if p + 1 < nump:
                cp = pltpu.async_copy(
                    table_hbm.at[idx_v.at[pl.ds((p + 1) * piece, piece)]],
                    bufs[1 - cur], sems[1 - cur])
            pltpu.sync_copy(bufs[cur], out_hbm.at[pl.ds(base + p * piece, piece)])

    return gather_kernel(table, idx_flat)


@jax.jit
def kernel(xyz, xyz_prev, features, features_prev, W1, b1):
    xyzt = jnp.transpose(xyz, (0, 2, 1))
    w1a = W1[:, :C_PREV]
    w1b = W1[:, C_PREV:]
    b1c = b1[:, None]

    grid = (B, N // TILE_N)
    iout, wout = pl.pallas_call(
        _knn_body,
        grid=grid,
        in_specs=[
            pl.BlockSpec((1, 3, TILE_N), lambda b, t: (b, 0, t)),
            pl.BlockSpec((1, M, 3), lambda b, t: (b, 0, 0)),
        ],
        out_specs=[
            pl.BlockSpec((1, 3, TILE_N), lambda b, t: (b, 0, t)),
            pl.BlockSpec((1, TILE_N, 3), lambda b, t: (b, t, 0)),
        ],
        out_shape=[
            jax.ShapeDtypeStruct((B, 3, N), jnp.int32),
            jax.ShapeDtypeStruct((B, N, 3), jnp.float32),
        ],
    )(xyzt, xyz_prev)

    table = jnp.transpose(features_prev, (0, 2, 1)).reshape(B * M, C_PREV)
    table = jnp.pad(table, ((0, 0), (0, 128 - C_PREV)))
    g_flat = _sc_gather(table, iout.reshape(-1))
    g = g_flat.reshape(B, 3, N, 128)

    out = pl.pallas_call(
        _mlp_body,
        grid=grid,
        in_specs=[
            pl.BlockSpec((1, 3, TILE_N, 128), lambda b, t: (b, 0, t, 0)),
            pl.BlockSpec((1, TILE_N, 3), lambda b, t: (b, t, 0)),
            pl.BlockSpec((1, C_FEAT, TILE_N), lambda b, t: (b, 0, t)),
            pl.BlockSpec((C_OUT, C_PREV), lambda b, t: (0, 0)),
            pl.BlockSpec((C_OUT, C_FEAT), lambda b, t: (0, 0)),
            pl.BlockSpec((C_OUT, 1), lambda b, t: (0, 0)),
        ],
        out_specs=pl.BlockSpec((1, C_OUT, TILE_N), lambda b, t: (b, 0, t)),
        out_shape=jax.ShapeDtypeStruct((B, C_OUT, N), jnp.float32),
    )(g, wout, features, w1a, w1b, b1c)
    return out


# speedup vs baseline: 1.0553x; 1.0553x over previous
import functools

import jax
import jax.numpy as jnp
from jax import lax
from jax.experimental import pallas as pl
from jax.experimental.pallas import tpu as pltpu
from jax.experimental.pallas import tpu_sc as plsc

B, N, M = 4, 16384, 1024
C_FEAT = 64
C_PREV = 64
C_OUT = 64

TILE_N = 1024


def _knn_body(xyzt_ref, xyzp_ref, iout_ref, wout_ref):
    q = xyzt_ref[...]
    k = xyzp_ref[...]

    d2 = jnp.zeros((M, TILE_N), jnp.float32)
    for c in range(3):
        diff = k[:, c:c + 1] - q[c:c + 1, :]
        d2 = d2 + diff * diff

    iota = jax.lax.broadcasted_iota(jnp.int32, (M, TILE_N), 0)
    bits = jax.lax.bitcast_convert_type(d2, jnp.int32) + (M // 2)
    keys = (bits & ~(M - 1)) | iota

    kcur = keys
    idxs = []
    invs = []
    tot = jnp.zeros((1, TILE_N), jnp.float32)
    for kk in range(3):
        fcur = jax.lax.bitcast_convert_type(kcur, jnp.float32)
        mkf = jnp.min(fcur, axis=0, keepdims=True)
        mk = jax.lax.bitcast_convert_type(mkf, jnp.int32)
        d2k = jax.lax.bitcast_convert_type(mk & ~(M - 1), jnp.float32)
        inv = 1.0 / (jnp.sqrt(d2k) + 1e-8)
        tot = tot + inv
        idxs.append(mk & (M - 1))
        invs.append(inv)
        if kk < 2:
            kcur = jnp.where(kcur == mk, jnp.int32(0x7F7FFFFF), kcur)

    iout_ref[...] = jnp.concatenate(idxs, axis=0)
    rtot = 1.0 / tot
    wcat = jnp.concatenate([w * rtot for w in invs], axis=0)
    wout_ref[...] = jnp.transpose(wcat)


def _knn_call(xyzt_b, xyzp_b):
    return pl.pallas_call(
        _knn_body,
        grid=(N // TILE_N,),
        in_specs=[
            pl.BlockSpec((3, TILE_N), lambda t: (0, t)),
            pl.BlockSpec((M, 3), lambda t: (0, 0)),
        ],
        out_specs=[
            pl.BlockSpec((3, TILE_N), lambda t: (0, t)),
            pl.BlockSpec((TILE_N, 3), lambda t: (t, 0)),
        ],
        out_shape=[
            jax.ShapeDtypeStruct((3, N), jnp.int32),
            jax.ShapeDtypeStruct((N, 3), jnp.float32),
        ],
    )(xyzt_b, xyzp_b)


def _mlp_body(g_ref, wt_ref, feat_ref, w1a_ref, w1b_ref, b1_ref, out_ref):
    wt = wt_ref[...]
    interp = g_ref[0, :, :C_PREV] * wt[:, 0:1]
    interp = interp + g_ref[1, :, :C_PREV] * wt[:, 1:2]
    interp = interp + g_ref[2, :, :C_PREV] * wt[:, 2:3]
    h = (jax.lax.dot_general(w1a_ref[...], interp, (((1,), (1,)), ((), ())),
                             preferred_element_type=jnp.float32)
         + jnp.dot(w1b_ref[...], feat_ref[...], preferred_element_type=jnp.float32)
         + b1_ref[...])
    out_ref[...] = jnp.maximum(h, 0.0)


def _mlp_call(g_b, wout_b, feat_b, w1a, w1b, b1c):
    return pl.pallas_call(
        _mlp_body,
        grid=(N // TILE_N,),
        in_specs=[
            pl.BlockSpec((3, TILE_N, 128), lambda t: (0, t, 0)),
            pl.BlockSpec((TILE_N, 3), lambda t: (t, 0)),
            pl.BlockSpec((C_FEAT, TILE_N), lambda t: (0, t)),
            pl.BlockSpec((C_OUT, C_PREV), lambda t: (0, 0)),
            pl.BlockSpec((C_OUT, C_FEAT), lambda t: (0, 0)),
            pl.BlockSpec((C_OUT, 1), lambda t: (0, 0)),
        ],
        out_specs=pl.BlockSpec((C_OUT, TILE_N), lambda t: (0, t)),
        out_shape=jax.ShapeDtypeStruct((C_OUT, N), jnp.float32),
    )(g_b, wout_b, feat_b, w1a, w1b, b1c)


def _sc_gather(table, idx_flat, total):
    info = plsc.get_sparse_core_info()
    nc, ns = info.num_cores, info.num_subcores
    nw = nc * ns
    chunk = total // nw
    piece = 384

    mesh = plsc.VectorSubcoreMesh(core_axis_name="c", subcore_axis_name="s")

    @functools.partial(
        pl.kernel, mesh=mesh,
        out_type=jax.ShapeDtypeStruct((total, 128), jnp.float32),
        scratch_types=[
            pltpu.VMEM((chunk,), jnp.int32),
            pltpu.VMEM((piece, 128), jnp.float32),
            pltpu.VMEM((piece, 128), jnp.float32),
            pltpu.SemaphoreType.DMA,
            pltpu.SemaphoreType.DMA,
        ],
    )
    def gather_kernel(table_hbm, idx_hbm, out_hbm, idx_v, rows0, rows1, sem0, sem1):
        wid = lax.axis_index("s") * nc + lax.axis_index("c")
        base = wid * chunk
        pltpu.sync_copy(idx_hbm.at[pl.ds(base, chunk)], idx_v)
        bufs = (rows0, rows1)
        sems = (sem0, sem1)
        nump = chunk // piece
        cp = pltpu.async_copy(
            table_hbm.at[idx_v.at[pl.ds(0, piece)]], bufs[0], sems[0])
        for p in range(nump):
            cur = p % 2
            cp.wait()
            if p + 1 < nump:
                cp = pltpu.async_copy(
                    table_hbm.at[idx_v.at[pl.ds((p + 1) * piece, piece)]],
                    bufs[1 - cur], sems[1 - cur])
            pltpu.sync_copy(bufs[cur], out_hbm.at[pl.ds(base + p * piece, piece)])

    return gather_kernel(table, idx_flat)


@jax.jit
def kernel(xyz, xyz_prev, features, features_prev, W1, b1):
    xyzt = jnp.transpose(xyz, (0, 2, 1))
    w1a = W1[:, :C_PREV]
    w1b = W1[:, C_PREV:]
    b1c = b1[:, None]

    table = jnp.transpose(features_prev, (0, 2, 1)).reshape(B * M, C_PREV)
    table = jnp.pad(table, ((0, 0), (0, 128 - C_PREV)))

    outs = []
    for b in range(B):
        iout_b, wout_b = _knn_call(xyzt[b], xyz_prev[b])
        idx_b = iout_b.reshape(-1) + b * M
        g_b = _sc_gather(table, idx_b, 3 * N).reshape(3, N, 128)
        outs.append(_mlp_call(g_b, wout_b, features[b], w1a, w1b, b1c))
    return jnp.stack(outs)
